# baseline (speedup 1.0000x reference)
import functools

import jax
import jax.numpy as jnp
from jax import lax
from jax.experimental import pallas as pl
from jax.experimental.pallas import tpu as pltpu
from jax.experimental.pallas import tpu_sc as plsc

_LANES = 16
_NUM_WORKERS = 32


def _make_gather(batch, vocab, hist, num_tables, embed_dim, slab, pack):
    num_rows = batch * num_tables * hist
    rows_per_w = num_rows // _NUM_WORKERS
    b_per_w = batch // _NUM_WORKERS
    g_per_dma = 128
    dmas_per_chunk = 10
    chunk = g_per_dma * dmas_per_chunk
    n_chunks = rows_per_w // chunk
    assert rows_per_w % chunk == 0 and rows_per_w % _LANES == 0
    per_b = num_tables * hist

    mesh = plsc.VectorSubcoreMesh(core_axis_name="c", subcore_axis_name="s")

    @functools.partial(
        pl.kernel,
        mesh=mesh,
        out_type=jax.ShapeDtypeStruct((num_rows, embed_dim), jnp.float32),
        scratch_types=[
            pltpu.VMEM((num_tables, b_per_w, hist), jnp.int32),
            pltpu.VMEM((rows_per_w,), jnp.int32),
            pltpu.VMEM((2, chunk, embed_dim), jnp.float32),
            pltpu.SemaphoreType.DMA,
            pltpu.SemaphoreType.DMA,
        ],
        compiler_params=pltpu.CompilerParams(
            use_tc_tiling_on_sc=False, needs_layout_passes=False
        ),
    )
    def gather_kernel(
        tab_hbm, idx_hbm, out_hbm, idx_s, idx_b, rows_v, gsem, osem
    ):
        wid = lax.axis_index("s") * 2 + lax.axis_index("c")
        base = pl.multiple_of(wid * rows_per_w, 8)
        b0 = pl.multiple_of(wid * b_per_w, 8)

        pltpu.sync_copy(idx_hbm.at[:, pl.ds(b0, b_per_w), :], idx_s)

        iota = lax.iota(jnp.int32, _LANES)
        tiles_per_b8 = per_b * embed_dim // 128
        rows_per_b8 = tiles_per_b8 * 32
        c_bb = jnp.full((_LANES,), rows_per_b8, jnp.int32)
        c_32 = jnp.full((_LANES,), 32, jnp.int32)
        c_4 = jnp.full((_LANES,), 4, jnp.int32)
        hist_v = jnp.full((_LANES,), hist, jnp.int32)
        c_slab = jnp.full((_LANES,), slab, jnp.int32)

        vecs_per_chunk = chunk // _LANES

        def reorder_body(j, _):
            off = pl.multiple_of(j * _LANES, 8)
            q = j * _LANES + iota
            bB = lax.div(q, c_bb)
            q1 = lax.rem(q, c_bb)
            ct = lax.div(q1, c_32)
            q2 = lax.rem(q1, c_32)
            r = lax.div(q2, c_4)
            m = lax.rem(q2, c_4)
            b_l = bB * 8 + r
            k = ct * 4 + m
            f = lax.div(k, hist_v)
            h = lax.rem(k, hist_v)
            v = plsc.load_gather(idx_s, [f, b_l, h])
            g = f * (slab * pack) + lax.rem(v, c_slab) * pack + lax.div(
                v, c_slab
            )
            idx_b[pl.ds(off, _LANES)] = g
            return _

        lax.fori_loop(0, vecs_per_chunk, reorder_body, 0)

        def chunk_body(c, _):
            buf = lax.rem(c, 2)
            crow = pl.multiple_of(c * chunk, 8)

            @pl.when(c >= 2)
            def _wait_prev_out():
                pltpu.make_async_copy(
                    rows_v.at[0], out_hbm.at[pl.ds(base, chunk)], osem
                ).wait()

            copies = []
            for g in range(dmas_per_chunk):
                src_idx = idx_b.at[pl.ds(crow + g * g_per_dma, g_per_dma)]
                dst = rows_v.at[buf, pl.ds(g * g_per_dma, g_per_dma)]
                copies.append(
                    pltpu.async_copy(tab_hbm.at[src_idx], dst, gsem)
                )

            @pl.when(c + 1 < n_chunks)
            def _reorder_next():
                lax.fori_loop(
                    (c + 1) * vecs_per_chunk,
                    (c + 2) * vecs_per_chunk,
                    reorder_body,
                    0,
                )

            for cp in copies:
                cp.wait()
            pltpu.async_copy(
                rows_v.at[buf], out_hbm.at[pl.ds(base + crow, chunk)], osem
            )
            return _

        lax.fori_loop(0, n_chunks, chunk_body, 0)
        for _ in range(2):
            pltpu.make_async_copy(
                rows_v.at[0], out_hbm.at[pl.ds(base, chunk)], osem
            ).wait()

    return gather_kernel


def _make_relayout(num_tables, vocab, embed_dim):
    pack = 128 // embed_dim
    slab = 25088
    vbk = 12544
    n_vb = slab // vbk

    def body(x0, x1, x2, x3, o_ref):
        o_ref[0] = jnp.concatenate(
            [jnp.transpose(xr[0]) for xr in (x0, x1, x2, x3)], axis=1
        )

    return pl.pallas_call(
        body,
        grid=(num_tables, n_vb),
        in_specs=[
            pl.BlockSpec(
                (1, embed_dim, vbk), lambda f, c, m=m: (f, 0, n_vb * m + c)
            )
            for m in range(pack)
        ],
        out_specs=pl.BlockSpec((1, vbk, 128), lambda f, c: (f, c, 0)),
        out_shape=jax.ShapeDtypeStruct(
            (num_tables, slab, 128), jnp.float32
        ),
    )


def kernel(indices, tables):
    num_tables, batch, hist = indices.shape
    _, vocab, embed_dim = tables.shape
    num_rows = batch * num_tables * hist

    pack = 128 // embed_dim
    slab = 25088
    tab_t = jnp.transpose(tables, (0, 2, 1))
    tab_rows = _make_relayout(num_tables, vocab, embed_dim)(
        tab_t, tab_t, tab_t, tab_t
    )
    tab_flat = tab_rows.reshape(num_tables * slab * pack, embed_dim)

    gather = _make_gather(
        batch, vocab, hist, num_tables, embed_dim, slab, pack
    )
    out = gather(tab_flat, indices.astype(jnp.int32))
    per_b = num_tables * hist
    n_bb = batch // 8
    n_ct = per_b * embed_dim // 128
    out = out.reshape(n_bb, n_ct, 8, 128)
    out = out.transpose(0, 2, 1, 3)
    return out.reshape(batch, per_b * embed_dim)


# speedup vs baseline: 1.0313x; 1.0313x over previous
import functools

import jax
import jax.numpy as jnp
from jax import lax
from jax.experimental import pallas as pl
from jax.experimental.pallas import tpu as pltpu
from jax.experimental.pallas import tpu_sc as plsc

_LANES = 16
_NUM_WORKERS = 32


def _make_gather(batch, vocab, hist, num_tables, embed_dim, slab, pack):
    num_rows = batch * num_tables * hist
    rows_per_w = num_rows // _NUM_WORKERS
    b_per_w = batch // _NUM_WORKERS
    g_per_dma = 128
    dmas_per_chunk = 10
    chunk = g_per_dma * dmas_per_chunk
    n_chunks = rows_per_w // chunk
    assert rows_per_w % chunk == 0 and rows_per_w % _LANES == 0
    per_b = num_tables * hist

    mesh = plsc.VectorSubcoreMesh(core_axis_name="c", subcore_axis_name="s")

    @functools.partial(
        pl.kernel,
        mesh=mesh,
        out_type=jax.ShapeDtypeStruct((num_rows, embed_dim), jnp.float32),
        scratch_types=[
            pltpu.VMEM((num_tables, b_per_w, hist), jnp.int32),
            pltpu.VMEM((rows_per_w,), jnp.int32),
            pltpu.VMEM((2, chunk, embed_dim), jnp.float32),
            pltpu.SemaphoreType.DMA,
            pltpu.SemaphoreType.DMA,
        ],
        compiler_params=pltpu.CompilerParams(
            use_tc_tiling_on_sc=False, needs_layout_passes=False
        ),
    )
    def gather_kernel(
        tab_hbm, idx_hbm, out_hbm, idx_s, idx_b, rows_v, gsem, osem
    ):
        wid = lax.axis_index("s") * 2 + lax.axis_index("c")
        base = pl.multiple_of(wid * rows_per_w, 8)
        b0 = pl.multiple_of(wid * b_per_w, 8)

        pltpu.sync_copy(idx_hbm.at[:, pl.ds(b0, b_per_w), :], idx_s)

        iota = lax.iota(jnp.int32, _LANES)
        tiles_per_b8 = per_b * embed_dim // 128
        rows_per_b8 = tiles_per_b8 * 32
        c_bb = jnp.full((_LANES,), rows_per_b8, jnp.int32)
        c_32 = jnp.full((_LANES,), 32, jnp.int32)
        c_4 = jnp.full((_LANES,), 4, jnp.int32)
        hist_v = jnp.full((_LANES,), hist, jnp.int32)
        c_slab = jnp.full((_LANES,), slab, jnp.int32)

        vecs_per_chunk = chunk // _LANES

        def reorder_body(j, _):
            off = pl.multiple_of(j * _LANES, 8)
            q = j * _LANES + iota
            bB = lax.div(q, c_bb)
            q1 = lax.rem(q, c_bb)
            ct = lax.div(q1, c_32)
            q2 = lax.rem(q1, c_32)
            r = lax.div(q2, c_4)
            m = lax.rem(q2, c_4)
            b_l = bB * 8 + r
            k = ct * 4 + m
            f = lax.div(k, hist_v)
            h = lax.rem(k, hist_v)
            v = plsc.load_gather(idx_s, [f, b_l, h])
            g = f * (slab * pack) + lax.rem(v, c_slab) * pack + lax.div(
                v, c_slab
            )
            idx_b[pl.ds(off, _LANES)] = g
            return _

        lax.fori_loop(0, rows_per_w // _LANES, reorder_body, 0)

        def chunk_body(c, _):
            buf = lax.rem(c, 2)
            crow = pl.multiple_of(c * chunk, 8)

            @pl.when(c >= 2)
            def _wait_prev_out():
                pltpu.make_async_copy(
                    rows_v.at[0], out_hbm.at[pl.ds(base, chunk)], osem
                ).wait()

            copies = []
            for g in range(dmas_per_chunk):
                src_idx = idx_b.at[pl.ds(crow + g * g_per_dma, g_per_dma)]
                dst = rows_v.at[buf, pl.ds(g * g_per_dma, g_per_dma)]
                copies.append(
                    pltpu.async_copy(tab_hbm.at[src_idx], dst, gsem)
                )

            for cp in copies:
                cp.wait()
            pltpu.async_copy(
                rows_v.at[buf], out_hbm.at[pl.ds(base + crow, chunk)], osem
            )
            return _

        lax.fori_loop(0, n_chunks, chunk_body, 0)
        for _ in range(2):
            pltpu.make_async_copy(
                rows_v.at[0], out_hbm.at[pl.ds(base, chunk)], osem
            ).wait()

    return gather_kernel


def _make_relayout(num_tables, vocab, embed_dim):
    pack = 128 // embed_dim
    slab = 25088
    vbk = 12544
    n_vb = slab // vbk

    def body(x0, x1, x2, x3, o_ref):
        o_ref[0] = jnp.concatenate(
            [jnp.transpose(xr[0]) for xr in (x0, x1, x2, x3)], axis=1
        )

    return pl.pallas_call(
        body,
        grid=(num_tables, n_vb),
        in_specs=[
            pl.BlockSpec(
                (1, embed_dim, vbk), lambda f, c, m=m: (f, 0, n_vb * m + c)
            )
            for m in range(pack)
        ],
        out_specs=pl.BlockSpec((1, vbk, 128), lambda f, c: (f, c, 0)),
        out_shape=jax.ShapeDtypeStruct(
            (num_tables, slab, 128), jnp.float32
        ),
    )


def kernel(indices, tables):
    num_tables, batch, hist = indices.shape
    _, vocab, embed_dim = tables.shape
    num_rows = batch * num_tables * hist

    pack = 128 // embed_dim
    slab = 25088
    tab_t = jnp.transpose(tables, (0, 2, 1))
    tab_rows = _make_relayout(num_tables, vocab, embed_dim)(
        tab_t, tab_t, tab_t, tab_t
    )
    tab_flat = tab_rows.reshape(num_tables * slab * pack, embed_dim)

    gather = _make_gather(
        batch, vocab, hist, num_tables, embed_dim, slab, pack
    )
    out = gather(tab_flat, indices.astype(jnp.int32))
    per_b = num_tables * hist
    n_bb = batch // 8
    n_ct = per_b * embed_dim // 128
    out = out.reshape(n_bb, n_ct, 8, 128)
    out = out.transpose(0, 2, 1, 3)
    return out.reshape(batch, per_b * embed_dim)
